# baseline (speedup 1.0000x reference)
import functools
import math

import numpy as np
import jax
import jax.numpy as jnp
from jax import lax
from jax.experimental import pallas as pl
from jax.experimental.pallas import tpu as pltpu
from jax.experimental.pallas import tpu_sc as plsc

_D_MODEL = 768
_MAX_LEN = 1600


def _make_pe_np():
    pe = np.zeros((_MAX_LEN, _D_MODEL), dtype=np.float32)
    position = np.arange(0, _MAX_LEN, dtype=np.float32)[:, None]
    div_term = np.exp(
        np.arange(0, _D_MODEL, 2, dtype=np.float32) * (-math.log(10000.0) / _D_MODEL)
    )
    pe[:, 0::2] = np.sin(position * div_term)
    pe[:, 1::2] = np.cos(position * div_term)
    return pe


def _dist_kernel(first_ref, out_ref):
    f = first_ref[:, 0, :]
    out_ref[...] = jnp.sum(jnp.abs(f[:, None, :] - f[None, :, :]), axis=-1)


def _splat_min(v):
    iota = lax.iota(jnp.int32, 16)
    dnums = lax.GatherDimensionNumbers(
        offset_dims=(), collapsed_slice_dims=(0,), start_index_map=(0,)
    )
    for k in (1, 2, 4, 8):
        perm = jnp.bitwise_and(iota + k, 15)[:, None]
        rot = lax.gather(
            v, perm, dnums, (1,), mode=lax.GatherScatterMode.PROMISE_IN_BOUNDS
        )
        v = jnp.minimum(v, rot)
    return v


def _make_top3_sc(sentence):
    info = plsc.get_sparse_core_info()
    nc, ns = info.num_cores, info.num_subcores
    nw = nc * ns
    rounds = -(-sentence // nw)
    mesh = plsc.VectorSubcoreMesh(core_axis_name="c", subcore_axis_name="s")

    @functools.partial(
        pl.kernel,
        mesh=mesh,
        out_type=jax.ShapeDtypeStruct((sentence, 16), jnp.int32),
        scratch_types=[
            pltpu.VMEM((sentence,), jnp.float32),
            pltpu.VMEM((16,), jnp.int32),
        ],
    )
    def top3_sc(soft_hbm, out_hbm, row_v, out_v):
        wid = lax.axis_index("s") * nc + lax.axis_index("c")
        iota = lax.iota(jnp.int32, 16)
        big = jnp.int32(10**6)
        inf = jnp.float32(jnp.inf)
        for t in range(rounds):
            r = wid + t * nw

            @pl.when(r < sentence)
            def _():
                pltpu.sync_copy(soft_hbm.at[r], row_v)
                lo = row_v[pl.ds(0, 16)]
                hi = row_v[pl.ds(16, 16)]
                picks = []
                for _k in range(3):
                    m = jnp.minimum(_splat_min(lo), _splat_min(hi))
                    ilo = _splat_min(jnp.where(lo == m, iota, big))
                    ihi = _splat_min(jnp.where(hi == m, iota + 16, big))
                    chosen = jnp.minimum(ilo, ihi)
                    picks.append(chosen)
                    lo = jnp.where(iota == chosen, inf, lo)
                    hi = jnp.where(iota + 16 == chosen, inf, hi)
                vec = jnp.where(
                    iota == 0,
                    picks[0],
                    jnp.where(iota == 1, picks[1], jnp.where(iota == 2, picks[2], 0)),
                )
                out_v[...] = vec
                pltpu.sync_copy(out_v, out_hbm.at[r])

    return top3_sc


def _attn_kernel(idx_ref, in0_ref, in1_ref, in2_ref, in3_ref, pe_ref, out_ref, xb_ref):
    i = pl.program_id(0)
    refs = (in0_ref, in1_ref, in2_ref, in3_ref)
    part_rows = in0_ref.shape[0]
    w = in0_ref.shape[1]
    n_per = out_ref.shape[0]
    h = in0_ref.shape[2]
    scale = 1.0 / math.sqrt(h)
    for j in range(n_per):
        for s in range(3):
            idxv = idx_ref[n_per * i + j, s]
            for part in range(4):

                @pl.when(jnp.logical_and(idxv >= part * part_rows,
                                         idxv < (part + 1) * part_rows))
                def _(part=part, j=j, s=s, idxv=idxv):
                    xs = refs[part][idxv - part * part_rows] + pe_ref[s]
                    xb_ref[j, s * w : (s + 1) * w] = xs.astype(jnp.bfloat16)
    for j in range(n_per):
        xb = xb_ref[j]
        q = xb[:w]
        scores = jax.lax.dot_general(
            q, xb, (((1,), (1,)), ((), ())), preferred_element_type=jnp.float32
        )
        scores = scores * scale
        m = jnp.max(scores, axis=1, keepdims=True)
        e = jnp.exp(scores - m)
        rinv = 1.0 / jnp.sum(e, axis=1, keepdims=True)
        eb = e.astype(jnp.bfloat16)
        out = jax.lax.dot_general(
            eb, xb, (((1,), (0,)), ((), ())), preferred_element_type=jnp.float32
        )
        out_ref[j] = (out * rinv)[: out_ref.shape[1]]


def kernel(inputs):
    sentence, word, hidden = inputs.shape

    soft = pl.pallas_call(
        _dist_kernel,
        grid=(1,),
        in_specs=[pl.BlockSpec((sentence, 8, hidden), lambda i: (0, 0, 0))],
        out_specs=pl.BlockSpec((sentence, sentence), lambda i: (0, 0)),
        out_shape=jax.ShapeDtypeStruct((sentence, sentence), jnp.float32),
    )(inputs)

    top3 = _make_top3_sc(sentence)(soft)

    pe3 = jnp.asarray(_make_pe_np()[: 3 * word].reshape(3, word, hidden))

    n_per = 8
    grid_spec = pltpu.PrefetchScalarGridSpec(
        num_scalar_prefetch=1,
        grid=(sentence // n_per,),
        in_specs=[
            pl.BlockSpec((sentence // 4, word, hidden), lambda i, idx: (0, 0, 0)),
            pl.BlockSpec((sentence // 4, word, hidden), lambda i, idx: (1, 0, 0)),
            pl.BlockSpec((sentence // 4, word, hidden), lambda i, idx: (2, 0, 0)),
            pl.BlockSpec((sentence // 4, word, hidden), lambda i, idx: (3, 0, 0)),
            pl.BlockSpec((3, word, hidden), lambda i, idx: (0, 0, 0)),
        ],
        out_specs=pl.BlockSpec((n_per, word - 1, hidden), lambda i, idx: (i, 0, 0)),
        scratch_shapes=[pltpu.VMEM((n_per, 3 * word, hidden), jnp.bfloat16)],
    )
    return pl.pallas_call(
        _attn_kernel,
        grid_spec=grid_spec,
        out_shape=jax.ShapeDtypeStruct((sentence, word - 1, hidden), jnp.float32),
    )(top3, inputs, inputs, inputs, inputs, pe3)


# speedup vs baseline: 1.0932x; 1.0932x over previous
import functools
import math

import numpy as np
import jax
import jax.numpy as jnp
from jax import lax
from jax.experimental import pallas as pl
from jax.experimental.pallas import tpu as pltpu
from jax.experimental.pallas import tpu_sc as plsc

_D_MODEL = 768
_MAX_LEN = 1600


def _make_pe_np():
    pe = np.zeros((_MAX_LEN, _D_MODEL), dtype=np.float32)
    position = np.arange(0, _MAX_LEN, dtype=np.float32)[:, None]
    div_term = np.exp(
        np.arange(0, _D_MODEL, 2, dtype=np.float32) * (-math.log(10000.0) / _D_MODEL)
    )
    pe[:, 0::2] = np.sin(position * div_term)
    pe[:, 1::2] = np.cos(position * div_term)
    return pe


def _dist_kernel(first_ref, out_ref):
    f = first_ref[:, 0, :]
    out_ref[...] = jnp.sum(jnp.abs(f[:, None, :] - f[None, :, :]), axis=-1)


def _splat_min(v):
    iota = lax.iota(jnp.int32, 16)
    dnums = lax.GatherDimensionNumbers(
        offset_dims=(), collapsed_slice_dims=(0,), start_index_map=(0,)
    )
    for k in (1, 2, 4, 8):
        perm = jnp.bitwise_and(iota + k, 15)[:, None]
        rot = lax.gather(
            v, perm, dnums, (1,), mode=lax.GatherScatterMode.PROMISE_IN_BOUNDS
        )
        v = jnp.minimum(v, rot)
    return v


def _make_top3_sc(sentence):
    info = plsc.get_sparse_core_info()
    nc, ns = info.num_cores, info.num_subcores
    nw = nc * ns
    rounds = -(-sentence // nw)
    mesh = plsc.VectorSubcoreMesh(core_axis_name="c", subcore_axis_name="s")

    @functools.partial(
        pl.kernel,
        mesh=mesh,
        out_type=jax.ShapeDtypeStruct((sentence, 16), jnp.int32),
        scratch_types=[
            pltpu.VMEM((sentence,), jnp.float32),
            pltpu.VMEM((16,), jnp.int32),
        ],
    )
    def top3_sc(soft_hbm, out_hbm, row_v, out_v):
        wid = lax.axis_index("s") * nc + lax.axis_index("c")
        iota = lax.iota(jnp.int32, 16)
        big = jnp.int32(10**6)
        inf = jnp.float32(jnp.inf)
        for t in range(rounds):
            r = wid + t * nw

            @pl.when(r < sentence)
            def _():
                pltpu.sync_copy(soft_hbm.at[r], row_v)
                lo = row_v[pl.ds(0, 16)]
                hi = row_v[pl.ds(16, 16)]
                picks = []
                for _k in range(3):
                    m = jnp.minimum(_splat_min(lo), _splat_min(hi))
                    ilo = _splat_min(jnp.where(lo == m, iota, big))
                    ihi = _splat_min(jnp.where(hi == m, iota + 16, big))
                    chosen = jnp.minimum(ilo, ihi)
                    picks.append(chosen)
                    lo = jnp.where(iota == chosen, inf, lo)
                    hi = jnp.where(iota + 16 == chosen, inf, hi)
                vec = jnp.where(
                    iota == 0,
                    picks[0],
                    jnp.where(iota == 1, picks[1], jnp.where(iota == 2, picks[2], 0)),
                )
                out_v[...] = vec
                pltpu.sync_copy(out_v, out_hbm.at[r])

    return top3_sc


def _attn_kernel(idx_ref, in_ref, pe_ref, out_ref, xb_ref):
    i = pl.program_id(0)
    w = in_ref.shape[1]
    n_per = out_ref.shape[0]
    h = in_ref.shape[2]
    scale = 1.0 / math.sqrt(h)
    for j in range(n_per):
        for s in range(3):
            xs = in_ref[idx_ref[n_per * i + j, s]] + pe_ref[s]
            xb_ref[j, s * w : (s + 1) * w] = xs.astype(jnp.bfloat16)
    for j in range(n_per):
        xb = xb_ref[j]
        q = xb[:w]
        scores = jax.lax.dot_general(
            q, xb, (((1,), (1,)), ((), ())), preferred_element_type=jnp.float32
        )
        scores = scores * scale
        m = jnp.max(scores, axis=1, keepdims=True)
        e = jnp.exp(scores - m)
        rinv = 1.0 / jnp.sum(e, axis=1, keepdims=True)
        eb = e.astype(jnp.bfloat16)
        out = jax.lax.dot_general(
            eb, xb, (((1,), (0,)), ((), ())), preferred_element_type=jnp.float32
        )
        out_ref[j] = (out * rinv)[: out_ref.shape[1]]


def kernel(inputs):
    sentence, word, hidden = inputs.shape

    soft = pl.pallas_call(
        _dist_kernel,
        grid=(1,),
        in_specs=[pl.BlockSpec((sentence, 8, hidden), lambda i: (0, 0, 0))],
        out_specs=pl.BlockSpec((sentence, sentence), lambda i: (0, 0)),
        out_shape=jax.ShapeDtypeStruct((sentence, sentence), jnp.float32),
    )(inputs)

    top3 = _make_top3_sc(sentence)(soft)

    pe3 = jnp.asarray(_make_pe_np()[: 3 * word].reshape(3, word, hidden))

    n_per = 8
    grid_spec = pltpu.PrefetchScalarGridSpec(
        num_scalar_prefetch=1,
        grid=(sentence // n_per,),
        in_specs=[
            pl.BlockSpec((sentence, word, hidden), lambda i, idx: (0, 0, 0)),
            pl.BlockSpec((3, word, hidden), lambda i, idx: (0, 0, 0)),
        ],
        out_specs=pl.BlockSpec((n_per, word - 1, hidden), lambda i, idx: (i, 0, 0)),
        scratch_shapes=[pltpu.VMEM((n_per, 3 * word, hidden), jnp.bfloat16)],
    )
    return pl.pallas_call(
        _attn_kernel,
        grid_spec=grid_spec,
        out_shape=jax.ShapeDtypeStruct((sentence, word - 1, hidden), jnp.float32),
    )(top3, inputs, pe3)
